# Initial kernel scaffold; baseline (speedup 1.0000x reference)
import jax
import jax.numpy as jnp
from jax.experimental import pallas as pl


def kernel(query, memory, adj_indices, Wq, bq, Wk, bk, Wv, bv, Wo, bo):
    raise NotImplementedError("write your pallas kernel here")


# speedup vs baseline: 3.1825x; 3.1825x over previous
import functools
import math

import jax
import jax.numpy as jnp
from jax import lax
from jax.experimental import pallas as pl
from jax.experimental.pallas import tpu as pltpu
from jax.experimental.pallas import tpu_sc as plsc

_NC = 2
_NS = 16
_NW = _NC * _NS

_HEADS = 4
_LANES = 16
_DLANE = 128


def _mesh():
    return plsc.VectorSubcoreMesh(core_axis_name="c", subcore_axis_name="s")


def _sc_gather(table, idx, chunk):
    e = idx.shape[0]
    d = table.shape[1]
    per_w = e // _NW
    nchunks = per_w // chunk

    @functools.partial(
        pl.kernel,
        mesh=_mesh(),
        out_type=jax.ShapeDtypeStruct((e, d), jnp.float32),
        scratch_types=[
            pltpu.VMEM((chunk,), jnp.int32),
            pltpu.VMEM((chunk, d), jnp.float32),
            pltpu.SemaphoreType.DMA,
        ],
    )
    def gk(table_hbm, idx_hbm, out_hbm, idx_v, rows_v, sem):
        wid = lax.axis_index("s") * _NC + lax.axis_index("c")
        base = wid * per_w

        def body(c, carry):
            off = base + c * chunk
            pltpu.sync_copy(idx_hbm.at[pl.ds(off, chunk)], idx_v)
            pltpu.async_copy(table_hbm.at[idx_v], rows_v, sem).wait()
            pltpu.sync_copy(rows_v, out_hbm.at[pl.ds(off, chunk)])
            return carry

        lax.fori_loop(0, nchunks, body, 0)

    return gk(table, idx)


def _sc_scatter_add(vals, idx, n_out, chunk):
    e, d = vals.shape
    per_w = e // _NW
    nchunks = per_w // chunk
    zeros = jnp.zeros((n_out, d), jnp.float32)

    @functools.partial(
        pl.kernel,
        mesh=_mesh(),
        out_type=jax.ShapeDtypeStruct((_NC, n_out, d), jnp.float32),
        scratch_types=[
            pltpu.VMEM((chunk,), jnp.int32),
            pltpu.VMEM((chunk, d), jnp.float32),
            pltpu.VMEM_SHARED((n_out, d), jnp.float32),
        ],
    )
    def sk(vals_hbm, idx_hbm, zeros_hbm, out_hbm, idx_v, vals_v, acc_sh):
        cid = lax.axis_index("c")
        sid = lax.axis_index("s")
        wid = sid * _NC + cid
        base = wid * per_w

        @pl.when(sid == 0)
        def _():
            pltpu.sync_copy(zeros_hbm, acc_sh)

        plsc.subcore_barrier()

        def body(c, carry):
            off = base + c * chunk
            pltpu.sync_copy(idx_hbm.at[pl.ds(off, chunk)], idx_v)
            pltpu.sync_copy(vals_hbm.at[pl.ds(off, chunk)], vals_v)
            pltpu.sync_copy(vals_v, acc_sh.at[idx_v], add=True)
            return carry

        lax.fori_loop(0, nchunks, body, 0)
        plsc.subcore_barrier()

        @pl.when(sid == 0)
        def _():
            pltpu.sync_copy(acc_sh, out_hbm.at[cid])

    return sk(vals, idx, zeros)


def _lrelu(x):
    return jnp.where(x >= 0, x, 0.2 * x)


def _tc_qkv(query, memory, wq, bq, wk, bk, wv, bv):
    n, d_in = query.shape
    hid = wq.shape[1]
    blk = 1000
    grid = n // blk

    def body(q_ref, m_ref, wq_ref, bq_ref, wk_ref, bk_ref, wv_ref, bv_ref,
             qo, ko, vo):
        x = q_ref[...]
        m = m_ref[...]
        hp = jax.lax.Precision.HIGHEST
        qq = jnp.dot(x, wq_ref[...], precision=hp,
                     preferred_element_type=jnp.float32) + bq_ref[...]
        kk = jnp.dot(m, wk_ref[...], precision=hp,
                     preferred_element_type=jnp.float32) + bk_ref[...]
        vv = jnp.dot(m, wv_ref[...], precision=hp,
                     preferred_element_type=jnp.float32) + bv_ref[...]
        qo[...] = _lrelu(qq)
        ko[...] = _lrelu(kk)
        vo[...] = _lrelu(vv)

    row_spec = pl.BlockSpec((blk, d_in), lambda i: (i, 0))
    w_spec = pl.BlockSpec((d_in, hid), lambda i: (0, 0))
    b_spec = pl.BlockSpec((1, hid), lambda i: (0, 0))
    out_spec = pl.BlockSpec((blk, hid), lambda i: (i, 0))
    shape = jax.ShapeDtypeStruct((n, hid), jnp.float32)
    return pl.pallas_call(
        body,
        grid=(grid,),
        in_specs=[row_spec, row_spec, w_spec, b_spec, w_spec, b_spec,
                  w_spec, b_spec],
        out_specs=[out_spec, out_spec, out_spec],
        out_shape=[shape, shape, shape],
    )(query, memory, wq, bq, wk, bk, wv, bv)


def _tc_dot(qe, ke, scale):
    e, hid = qe.shape
    hd = hid // _HEADS
    blk = 2000
    grid = e // blk

    def body(q_ref, k_ref, o_ref):
        p = q_ref[...] * k_ref[...]
        ds = [jnp.sum(p[:, h * hd:(h + 1) * hd], axis=1, keepdims=True) * scale
              for h in range(_HEADS)]
        d4 = jnp.concatenate(ds, axis=1)
        o_ref[...] = jnp.concatenate([d4] * (_DLANE // _HEADS), axis=1)

    return pl.pallas_call(
        body,
        grid=(grid,),
        in_specs=[pl.BlockSpec((blk, hid), lambda i: (i, 0)),
                  pl.BlockSpec((blk, hid), lambda i: (i, 0))],
        out_specs=pl.BlockSpec((blk, _DLANE), lambda i: (i, 0)),
        out_shape=jax.ShapeDtypeStruct((e, _DLANE), jnp.float32),
    )(qe, ke)


def _tc_maxexp(d16):
    e = d16.shape[0]
    blk = 8000
    nb = e // blk

    def body(d_ref, o_ref, mx_ref):
        p = pl.program_id(0)
        b = pl.program_id(1)

        @pl.when(jnp.logical_and(p == 0, b == 0))
        def _():
            mx_ref[0] = -jnp.inf

        @pl.when(p == 0)
        def _():
            mx_ref[0] = jnp.maximum(mx_ref[0], jnp.max(d_ref[...]))

        @pl.when(p == 1)
        def _():
            o_ref[...] = jnp.exp(d_ref[...] - mx_ref[0])

    return pl.pallas_call(
        body,
        grid=(2, nb),
        in_specs=[pl.BlockSpec((blk, _DLANE), lambda p, b: (b, 0))],
        out_specs=pl.BlockSpec((blk, _DLANE), lambda p, b: (b, 0)),
        out_shape=jax.ShapeDtypeStruct((e, _DLANE), jnp.float32),
        scratch_shapes=[pltpu.SMEM((1,), jnp.float32)],
    )(d16)


def _tc_weight(ex16, g0, g1, ve):
    e, hid = ve.shape
    od = hid // _HEADS
    blk = 2000
    grid = e // blk

    def body(ex_ref, g0_ref, g1_ref, v_ref, o_ref):
        w16 = ex_ref[...] / (g0_ref[...] + g1_ref[...])
        v = v_ref[...]
        parts = [w16[:, h:h + 1] * v[:, h * od:(h + 1) * od]
                 for h in range(_HEADS)]
        o_ref[...] = jnp.concatenate(parts, axis=1)

    lane_spec = pl.BlockSpec((blk, _DLANE), lambda i: (i, 0))
    return pl.pallas_call(
        body,
        grid=(grid,),
        in_specs=[lane_spec, lane_spec, lane_spec,
                  pl.BlockSpec((blk, hid), lambda i: (i, 0))],
        out_specs=pl.BlockSpec((blk, hid), lambda i: (i, 0)),
        out_shape=jax.ShapeDtypeStruct((e, hid), jnp.float32),
    )(ex16, g0, g1, ve)


def _tc_final(parts, wo, bo):
    n = parts[0].shape[1]
    od = parts[0].shape[2]
    out_d = wo.shape[1]
    blk = 1000
    grid = n // blk

    def body(p0_ref, p1_ref, p2_ref, p3_ref, wo_ref, bo_ref, o_ref):
        hp = jax.lax.Precision.HIGHEST
        w = wo_ref[...]
        acc = jnp.zeros((blk, out_d), jnp.float32)
        for h, pr in enumerate([p0_ref, p1_ref, p2_ref, p3_ref]):
            s = pr[0] + pr[1]
            acc = acc + jnp.dot(s, w[h * od:(h + 1) * od, :], precision=hp,
                                preferred_element_type=jnp.float32)
        o_ref[...] = acc + bo_ref[...]

    p_spec = pl.BlockSpec((_NC, blk, od), lambda i: (0, i, 0))
    return pl.pallas_call(
        body,
        grid=(grid,),
        in_specs=[p_spec, p_spec, p_spec, p_spec,
                  pl.BlockSpec(wo.shape, lambda i: (0, 0)),
                  pl.BlockSpec((1, out_d), lambda i: (0, 0))],
        out_specs=pl.BlockSpec((blk, out_d), lambda i: (i, 0)),
        out_shape=jax.ShapeDtypeStruct((n, out_d), jnp.float32),
    )(parts[0], parts[1], parts[2], parts[3], wo, bo)


def kernel(query, memory, adj_indices, Wq, bq, Wk, bk, Wv, bv, Wo, bo):
    n = query.shape[0]
    hid = Wq.shape[1]
    hd = hid // _HEADS
    od = Wv.shape[1] // _HEADS
    rows = adj_indices[:, 0]
    cols = adj_indices[:, 1]

    q, k, v = _tc_qkv(query, memory, Wq, bq.reshape(1, -1), Wk,
                      bk.reshape(1, -1), Wv, bv.reshape(1, -1))

    qe = _sc_gather(q, rows, 40)
    ke = _sc_gather(k, cols, 40)
    ve = _sc_gather(v, cols, 40)

    d16 = _tc_dot(qe, ke, 1.0 / math.sqrt(hd))
    ex16 = _tc_maxexp(d16)

    dpart = _sc_scatter_add(ex16, cols, n, 40)
    g0 = _sc_gather(dpart[0], cols, 40)
    g1 = _sc_gather(dpart[1], cols, 40)

    ye = _tc_weight(ex16, g0, g1, ve)

    ps = [_sc_scatter_add(ye[:, h * od:(h + 1) * od], rows, n, 40)
          for h in range(_HEADS)]

    return _tc_final(ps, Wo, bo.reshape(1, -1))


# speedup vs baseline: 3.8875x; 1.2215x over previous
import functools
import math

import jax
import jax.numpy as jnp
from jax import lax
from jax.experimental import pallas as pl
from jax.experimental.pallas import tpu as pltpu
from jax.experimental.pallas import tpu_sc as plsc

_NC = 2
_NS = 16
_NW = _NC * _NS

_HEADS = 4
_LANES = 16
_DLANE = 128


def _mesh():
    return plsc.VectorSubcoreMesh(core_axis_name="c", subcore_axis_name="s")


def _sc_gather(table, idx, chunk):
    e = idx.shape[0]
    d = table.shape[1]
    per_w = e // _NW
    nchunks = per_w // chunk

    @functools.partial(
        pl.kernel,
        mesh=_mesh(),
        out_type=jax.ShapeDtypeStruct((e, d), jnp.float32),
        scratch_types=[
            pltpu.VMEM((chunk,), jnp.int32),
            pltpu.VMEM((chunk, d), jnp.float32),
            pltpu.SemaphoreType.DMA,
        ],
    )
    def gk(table_hbm, idx_hbm, out_hbm, idx_v, rows_v, sem):
        wid = lax.axis_index("s") * _NC + lax.axis_index("c")
        base = wid * per_w

        def body(c, carry):
            off = base + c * chunk
            pltpu.sync_copy(idx_hbm.at[pl.ds(off, chunk)], idx_v)
            pltpu.async_copy(table_hbm.at[idx_v], rows_v, sem).wait()
            pltpu.sync_copy(rows_v, out_hbm.at[pl.ds(off, chunk)])
            return carry

        lax.fori_loop(0, nchunks, body, 0)

    return gk(table, idx)


def _sc_scatter_add(vals, idx, n_out, chunk):
    e, d = vals.shape
    per_w = e // _NW
    nchunks = per_w // chunk
    zeros = jnp.zeros((n_out, d), jnp.float32)

    @functools.partial(
        pl.kernel,
        mesh=_mesh(),
        out_type=jax.ShapeDtypeStruct((_NC, n_out, d), jnp.float32),
        scratch_types=[
            pltpu.VMEM((chunk,), jnp.int32),
            pltpu.VMEM((chunk, d), jnp.float32),
            pltpu.VMEM_SHARED((n_out, d), jnp.float32),
        ],
    )
    def sk(vals_hbm, idx_hbm, zeros_hbm, out_hbm, idx_v, vals_v, acc_sh):
        cid = lax.axis_index("c")
        sid = lax.axis_index("s")
        wid = sid * _NC + cid
        base = wid * per_w

        @pl.when(sid == 0)
        def _():
            pltpu.sync_copy(zeros_hbm, acc_sh)

        plsc.subcore_barrier()

        def body(c, carry):
            off = base + c * chunk
            pltpu.sync_copy(idx_hbm.at[pl.ds(off, chunk)], idx_v)
            pltpu.sync_copy(vals_hbm.at[pl.ds(off, chunk)], vals_v)
            pltpu.sync_copy(vals_v, acc_sh.at[idx_v], add=True)
            return carry

        lax.fori_loop(0, nchunks, body, 0)
        plsc.subcore_barrier()

        @pl.when(sid == 0)
        def _():
            pltpu.sync_copy(acc_sh, out_hbm.at[cid])

    return sk(vals, idx, zeros)


def _lrelu(x):
    return jnp.where(x >= 0, x, 0.2 * x)


def _tc_qkv(query, memory, wq, bq, wk, bk, wv, bv):
    n, d_in = query.shape
    hid = wq.shape[1]
    blk = 1000
    grid = n // blk

    def body(q_ref, m_ref, wq_ref, bq_ref, wk_ref, bk_ref, wv_ref, bv_ref,
             qo, ko, vo):
        x = q_ref[...]
        m = m_ref[...]
        hp = jax.lax.Precision.HIGHEST
        qq = jnp.dot(x, wq_ref[...], precision=hp,
                     preferred_element_type=jnp.float32) + bq_ref[...]
        kk = jnp.dot(m, wk_ref[...], precision=hp,
                     preferred_element_type=jnp.float32) + bk_ref[...]
        vv = jnp.dot(m, wv_ref[...], precision=hp,
                     preferred_element_type=jnp.float32) + bv_ref[...]
        qo[...] = _lrelu(qq)
        ko[...] = _lrelu(kk)
        vo[...] = _lrelu(vv)

    row_spec = pl.BlockSpec((blk, d_in), lambda i: (i, 0))
    w_spec = pl.BlockSpec((d_in, hid), lambda i: (0, 0))
    b_spec = pl.BlockSpec((1, hid), lambda i: (0, 0))
    out_spec = pl.BlockSpec((blk, hid), lambda i: (i, 0))
    shape = jax.ShapeDtypeStruct((n, hid), jnp.float32)
    return pl.pallas_call(
        body,
        grid=(grid,),
        in_specs=[row_spec, row_spec, w_spec, b_spec, w_spec, b_spec,
                  w_spec, b_spec],
        out_specs=[out_spec, out_spec, out_spec],
        out_shape=[shape, shape, shape],
    )(query, memory, wq, bq, wk, bk, wv, bv)


def _tc_dot(qe, ke, scale):
    e, hid = qe.shape
    hd = hid // _HEADS
    blk = 2000
    grid = e // blk

    def body(q_ref, k_ref, o_ref):
        p = q_ref[...] * k_ref[...]
        ds = [jnp.sum(p[:, h * hd:(h + 1) * hd], axis=1, keepdims=True) * scale
              for h in range(_HEADS)]
        d4 = jnp.concatenate(ds, axis=1)
        o_ref[...] = jnp.concatenate([d4] * (_DLANE // _HEADS), axis=1)

    return pl.pallas_call(
        body,
        grid=(grid,),
        in_specs=[pl.BlockSpec((blk, hid), lambda i: (i, 0)),
                  pl.BlockSpec((blk, hid), lambda i: (i, 0))],
        out_specs=pl.BlockSpec((blk, _DLANE), lambda i: (i, 0)),
        out_shape=jax.ShapeDtypeStruct((e, _DLANE), jnp.float32),
    )(qe, ke)


def _tc_maxexp(d16):
    e = d16.shape[0]
    blk = 8000
    nb = e // blk

    def body(d_ref, o_ref, mx_ref):
        p = pl.program_id(0)
        b = pl.program_id(1)

        @pl.when(jnp.logical_and(p == 0, b == 0))
        def _():
            mx_ref[0] = -jnp.inf

        @pl.when(p == 0)
        def _():
            mx_ref[0] = jnp.maximum(mx_ref[0], jnp.max(d_ref[...]))

        @pl.when(p == 1)
        def _():
            o_ref[...] = jnp.exp(d_ref[...] - mx_ref[0])

    return pl.pallas_call(
        body,
        grid=(2, nb),
        in_specs=[pl.BlockSpec((blk, _DLANE), lambda p, b: (b, 0))],
        out_specs=pl.BlockSpec((blk, _DLANE), lambda p, b: (b, 0)),
        out_shape=jax.ShapeDtypeStruct((e, _DLANE), jnp.float32),
        scratch_shapes=[pltpu.SMEM((1,), jnp.float32)],
    )(d16)


def _tc_weight(ex16, g0, g1, ve):
    e, hid = ve.shape
    od = hid // _HEADS
    blk = 2000
    grid = e // blk

    def body(ex_ref, g0_ref, g1_ref, v_ref, o_ref):
        w16 = ex_ref[...] / (g0_ref[...] + g1_ref[...])
        v = v_ref[...]
        parts = [w16[:, h:h + 1] * v[:, h * od:(h + 1) * od]
                 for h in range(_HEADS)]
        o_ref[...] = jnp.concatenate(parts, axis=1)

    lane_spec = pl.BlockSpec((blk, _DLANE), lambda i: (i, 0))
    return pl.pallas_call(
        body,
        grid=(grid,),
        in_specs=[lane_spec, lane_spec, lane_spec,
                  pl.BlockSpec((blk, hid), lambda i: (i, 0))],
        out_specs=pl.BlockSpec((blk, hid), lambda i: (i, 0)),
        out_shape=jax.ShapeDtypeStruct((e, hid), jnp.float32),
    )(ex16, g0, g1, ve)


def _tc_final(parts, wo, bo):
    n = parts[0].shape[1]
    od = parts[0].shape[2]
    out_d = wo.shape[1]
    blk = 1000
    grid = n // blk

    def body(p0_ref, p1_ref, p2_ref, p3_ref, wo_ref, bo_ref, o_ref):
        hp = jax.lax.Precision.HIGHEST
        w = wo_ref[...]
        acc = jnp.zeros((blk, out_d), jnp.float32)
        for h, pr in enumerate([p0_ref, p1_ref, p2_ref, p3_ref]):
            s = pr[0] + pr[1]
            acc = acc + jnp.dot(s, w[h * od:(h + 1) * od, :], precision=hp,
                                preferred_element_type=jnp.float32)
        o_ref[...] = acc + bo_ref[...]

    p_spec = pl.BlockSpec((_NC, blk, od), lambda i: (0, i, 0))
    return pl.pallas_call(
        body,
        grid=(grid,),
        in_specs=[p_spec, p_spec, p_spec, p_spec,
                  pl.BlockSpec(wo.shape, lambda i: (0, 0)),
                  pl.BlockSpec((1, out_d), lambda i: (0, 0))],
        out_specs=pl.BlockSpec((blk, out_d), lambda i: (i, 0)),
        out_shape=jax.ShapeDtypeStruct((n, out_d), jnp.float32),
    )(parts[0], parts[1], parts[2], parts[3], wo, bo)


def kernel(query, memory, adj_indices, Wq, bq, Wk, bk, Wv, bv, Wo, bo):
    n = query.shape[0]
    hid = Wq.shape[1]
    hd = hid // _HEADS
    od = Wv.shape[1] // _HEADS
    rows = adj_indices[:, 0]
    cols = adj_indices[:, 1]

    q, k, v = _tc_qkv(query, memory, Wq, bq.reshape(1, -1), Wk,
                      bk.reshape(1, -1), Wv, bv.reshape(1, -1))

    qe = _sc_gather(q, rows, 40)
    ke = _sc_gather(k, cols, 40)
    ve = _sc_gather(v, cols, 40)

    d16 = _tc_dot(qe, ke, 1.0 / math.sqrt(hd))
    ex16 = _tc_maxexp(d16)

    dpart = _sc_scatter_add(ex16, cols, n, 200)
    g0 = _sc_gather(dpart[0], cols, 200)
    g1 = _sc_gather(dpart[1], cols, 200)

    ye = _tc_weight(ex16, g0, g1, ve)

    ps = [_sc_scatter_add(ye[:, h * od:(h + 1) * od], rows, n, 200)
          for h in range(_HEADS)]

    return _tc_final(ps, Wo, bo.reshape(1, -1))
